# baseline (speedup 1.0000x reference)
import functools

import jax
import jax.numpy as jnp
from jax import lax
from jax.experimental import pallas as pl
from jax.experimental.pallas import tpu as pltpu
from jax.experimental.pallas import tpu_sc as plsc

N = 10000
E = 320000
D = 128
NITER = 3
G = 128

NP = 10240
RB = 1024
NBLK = NP // RB

NSC = 2
NTILE = 16
K = 125
TPW = E // (NSC * NTILE)
CH = TPW // K
NBUF = 4
GRPS = CH // NBUF
ZR = NP // NTILE

NEG = -1e30
_P = lax.Precision.HIGHEST


def _dot(x, y, dn=None):
    if dn is None:
        return lax.dot(x, y, precision=_P, preferred_element_type=jnp.float32)
    return lax.dot_general(x, y, dn, precision=_P,
                           preferred_element_type=jnp.float32)


def _make_sc_segsum(width, gather):
    mesh = plsc.VectorSubcoreMesh(core_axis_name="c", subcore_axis_name="s",
                                  num_cores=NSC, num_subcores=NTILE)
    if width is None:
        row_shape = (K,)
        acc_words = NP
        out_shape = (NSC, NP)
        nbuf = 8
        ib = CH
    else:
        row_shape = (K, width)
        acc_words = NP * width
        out_shape = (NSC, NP, width)
        nbuf = 2
        ib = 16
    nib = CH // ib
    acc_shape = (NP,) if width is None else (NP, width)
    scratch = [
        pltpu.VMEM((ib, K), jnp.int32),
        *[pltpu.VMEM(row_shape, jnp.float32) for _ in range(nbuf)],
        pltpu.VMEM_SHARED(acc_shape, jnp.float32),
        pltpu.SemaphoreType.DMA((nbuf,)),
        pltpu.SemaphoreType.DMA((nbuf,)),
    ]
    if gather:
        scratch.insert(0, pltpu.VMEM((ib, K), jnp.int32))

    @functools.partial(
        pl.kernel,
        out_type=jax.ShapeDtypeStruct(out_shape, jnp.float32),
        mesh=mesh,
        scratch_types=scratch,
    )
    def kern(*refs):
        if gather:
            (table_hbm, src_hbm, dst_hbm, zrow_hbm, out_hbm,
             sidx_v, didx_v, *rest) = refs
        else:
            (ones_hbm, dst_hbm, zrow_hbm, out_hbm, didx_v, *rest) = refs
        bufs = rest[:nbuf]
        acc_sh, gsem, ssem = rest[nbuf:]
        cid = lax.axis_index("c")
        sid = lax.axis_index("s")
        wid = cid * NTILE + sid
        if not gather:
            for bb in range(nbuf):
                pltpu.sync_copy(ones_hbm, bufs[bb])
        pltpu.sync_copy(zrow_hbm, acc_sh.at[pl.ds(sid * ZR, ZR)])
        plsc.subcore_barrier()

        def stage_idx(nb):
            off = wid * CH + nb * ib
            pltpu.sync_copy(dst_hbm.at[pl.ds(off, ib)], didx_v)
            if gather:
                pltpu.sync_copy(src_hbm.at[pl.ds(off, ib)], sidx_v)

        def start_gather(j, b):
            return pltpu.async_copy(table_hbm.at[sidx_v.at[j]], bufs[b],
                                    gsem.at[b])

        def start_scatter(j, b):
            return pltpu.async_copy(bufs[b], acc_sh.at[didx_v.at[j]],
                                    ssem.at[b], add=True)

        def wait_gather(b):
            pltpu.make_async_copy(table_hbm.at[sidx_v.at[0]], bufs[b],
                                  gsem.at[b]).wait()

        def wait_scatter(b):
            pltpu.make_async_copy(bufs[b], acc_sh.at[didx_v.at[0]],
                                  ssem.at[b]).wait()

        if gather and width is not None:
            def blk(nb, carry):
                stage_idx(nb)
                start_gather(0, 0)

                def body(t, c):
                    j = 2 * t
                    wait_gather(0)
                    start_scatter(j, 0)
                    start_gather(j + 1, 1)
                    wait_scatter(0)
                    wait_gather(1)
                    start_scatter(j + 1, 1)
                    start_gather(jnp.minimum(j + 2, ib - 1), 0)
                    wait_scatter(1)
                    return c

                lax.fori_loop(0, ib // 2, body, 0)
                wait_gather(0)
                return carry

            lax.fori_loop(0, nib, blk, 0)
        elif gather:
            stage_idx(0)
            for bb in range(nbuf):
                start_gather(bb, bb)

            def body(g, carry):
                j0 = g * nbuf
                for bb in range(nbuf):
                    wait_gather(bb)
                    start_scatter(j0 + bb, bb)
                for bb in range(nbuf):
                    wait_scatter(bb)
                    start_gather(jnp.minimum(j0 + nbuf + bb, CH - 1), bb)
                return carry

            lax.fori_loop(0, CH // nbuf, body, 0)
            for bb in range(nbuf):
                wait_gather(bb)
        else:
            stage_idx(0)

            def body(g, carry):
                j0 = g * nbuf
                for bb in range(nbuf):
                    start_scatter(j0 + bb, bb)
                for bb in range(nbuf):
                    wait_scatter(bb)
                return carry

            lax.fori_loop(0, CH // nbuf, body, 0)
        plsc.subcore_barrier()
        pltpu.sync_copy(acc_sh.at[pl.ds(sid * ZR, ZR)],
                        out_hbm.at[cid, pl.ds(sid * ZR, ZR)])

    return kern


def _make_sc_combo():
    mesh = plsc.VectorSubcoreMesh(core_axis_name="c", subcore_axis_name="s",
                                  num_cores=NSC, num_subcores=NTILE)
    ib = 16
    nib = CH // ib
    scratch = [
        pltpu.VMEM((ib, K), jnp.int32),
        pltpu.VMEM((ib, K), jnp.int32),
        pltpu.VMEM((K, D), jnp.float32),
        pltpu.VMEM((K, D), jnp.float32),
        pltpu.VMEM((K,), jnp.float32),
        pltpu.VMEM((K,), jnp.float32),
        pltpu.VMEM_SHARED((NP, D), jnp.float32),
        pltpu.VMEM_SHARED((NP,), jnp.float32),
        pltpu.SemaphoreType.DMA((2,)),
        pltpu.SemaphoreType.DMA((2,)),
        pltpu.SemaphoreType.DMA((2,)),
        pltpu.SemaphoreType.DMA((2,)),
    ]

    @functools.partial(
        pl.kernel,
        out_type=(jax.ShapeDtypeStruct((NSC, NP, D), jnp.float32),
                  jax.ShapeDtypeStruct((NSC, NP), jnp.float32)),
        mesh=mesh,
        scratch_types=scratch,
    )
    def kern(u_hbm, y_hbm, src_hbm, dst_hbm, zrow_hbm, zs_hbm,
             outr_hbm, outs_hbm, sidx_v, didx_v, r0, r1, s0, s1,
             accr, accs, gsr, ssr, gss, sss):
        rbufs = (r0, r1)
        sbufs = (s0, s1)
        cid = lax.axis_index("c")
        sid = lax.axis_index("s")
        wid = cid * NTILE + sid
        pltpu.sync_copy(zrow_hbm, accr.at[pl.ds(sid * ZR, ZR)])
        pltpu.sync_copy(zs_hbm, accs.at[pl.ds(sid * ZR, ZR)])
        plsc.subcore_barrier()

        def gath_r(j, b):
            pltpu.async_copy(u_hbm.at[sidx_v.at[j]], rbufs[b], gsr.at[b])

        def scat_r(j, b):
            pltpu.async_copy(rbufs[b], accr.at[didx_v.at[j]], ssr.at[b],
                             add=True)

        def gath_s(j, b):
            pltpu.async_copy(y_hbm.at[sidx_v.at[j]], sbufs[b], gss.at[b])

        def scat_s(j, b):
            pltpu.async_copy(sbufs[b], accs.at[didx_v.at[j]], sss.at[b],
                             add=True)

        def wg_r(b):
            pltpu.make_async_copy(u_hbm.at[sidx_v.at[0]], rbufs[b],
                                  gsr.at[b]).wait()

        def ws_r(b):
            pltpu.make_async_copy(rbufs[b], accr.at[didx_v.at[0]],
                                  ssr.at[b]).wait()

        def wg_s(b):
            pltpu.make_async_copy(y_hbm.at[sidx_v.at[0]], sbufs[b],
                                  gss.at[b]).wait()

        def ws_s(b):
            pltpu.make_async_copy(sbufs[b], accs.at[didx_v.at[0]],
                                  sss.at[b]).wait()

        def blk(nb, carry):
            off = wid * CH + nb * ib
            pltpu.sync_copy(dst_hbm.at[pl.ds(off, ib)], didx_v)
            pltpu.sync_copy(src_hbm.at[pl.ds(off, ib)], sidx_v)
            gath_r(0, 0)
            gath_s(0, 0)

            def body(t, c):
                j = 2 * t
                wg_r(0)
                scat_r(j, 0)
                gath_r(j + 1, 1)
                wg_s(0)
                scat_s(j, 0)
                gath_s(j + 1, 1)
                ws_r(0)
                ws_s(0)
                wg_r(1)
                scat_r(j + 1, 1)
                gath_r(jnp.minimum(j + 2, ib - 1), 0)
                wg_s(1)
                scat_s(j + 1, 1)
                gath_s(jnp.minimum(j + 2, ib - 1), 0)
                ws_r(1)
                ws_s(1)
                return c

            lax.fori_loop(0, ib // 2, body, 0)
            wg_r(0)
            wg_s(0)
            return carry

        lax.fori_loop(0, nib, blk, 0)
        plsc.subcore_barrier()
        pltpu.sync_copy(accr.at[pl.ds(sid * ZR, ZR)],
                        outr_hbm.at[cid, pl.ds(sid * ZR, ZR)])
        pltpu.sync_copy(accs.at[pl.ds(sid * ZR, ZR)],
                        outs_hbm.at[cid, pl.ds(sid * ZR, ZR)])

    return kern


_make_sc_combo = functools.lru_cache(maxsize=None)(_make_sc_combo)


_make_sc_segsum = functools.lru_cache(maxsize=None)(_make_sc_segsum)


def _sc_rows(table, src, dst, zrow):
    return _make_sc_segsum(D, True)(table, src, dst, zrow)


def _sc_scalar(table, src, dst, zrow):
    return _make_sc_segsum(None, True)(table, src, dst, zrow)


def _sc_deg(ones, dst, zrow):
    return _make_sc_segsum(None, False)(ones, dst, zrow)


def _pre_body(g_ref, w_ref, d0_ref, d1_ref, u_ref):
    deg = d0_ref[...] + d1_ref[...] + 1.0
    dis = lax.rsqrt(deg)
    u_ref[...] = _dot(g_ref[...], w_ref[...]) * dis


def _pre_call(g, w, d0, d1):
    return pl.pallas_call(
        _pre_body,
        grid=(NBLK,),
        in_specs=[
            pl.BlockSpec((RB, D), lambda i: (i, 0)),
            pl.BlockSpec((D, D), lambda i: (0, 0)),
            pl.BlockSpec((RB, 1), lambda i: (i, 0)),
            pl.BlockSpec((RB, 1), lambda i: (i, 0)),
        ],
        out_specs=pl.BlockSpec((RB, D), lambda i: (i, 0)),
        out_shape=jax.ShapeDtypeStruct((NP, D), jnp.float32),
    )(g, w, d0, d1)


def _post_body(a0_ref, a1_ref, u_ref, ea_ref, d0_ref, d1_ref, b_ref,
               wrel_ref, wroot_ref, wnext_ref, g_ref, y_ref, r_ref, un_ref):
    deg = d0_ref[...] + d1_ref[...] + 1.0
    dis = lax.rsqrt(deg)
    agg = a0_ref[0] + a1_ref[0]
    g = ea_ref[...] + dis * (agg + u_ref[...]) + b_ref[...]
    g = jnp.maximum(g, 0.0)
    rows = pl.program_id(0) * RB + lax.broadcasted_iota(jnp.int32, (RB, 1), 0)
    g = jnp.where(rows < N, g, 0.0)
    g_ref[...] = g
    y_ref[...] = _dot(g, wrel_ref[...])
    r_ref[...] = _dot(g, wroot_ref[...])
    un_ref[...] = _dot(g, wnext_ref[...]) * dis


def _post_call(aggp, u, ea, d0, d1, b, wrel, wroot, wnext):
    blk = pl.BlockSpec((RB, D), lambda i: (i, 0))
    col = pl.BlockSpec((RB, 1), lambda i: (i, 0))
    return pl.pallas_call(
        _post_body,
        grid=(NBLK,),
        in_specs=[
            pl.BlockSpec((1, RB, D), lambda i: (0, i, 0)),
            pl.BlockSpec((1, RB, D), lambda i: (1, i, 0)),
            blk, blk, col, col,
            pl.BlockSpec((1, D), lambda i: (0, 0)),
            pl.BlockSpec((D, 1), lambda i: (0, 0)),
            pl.BlockSpec((D, 1), lambda i: (0, 0)),
            pl.BlockSpec((D, D), lambda i: (0, 0)),
        ],
        out_specs=[blk, col, col, blk],
        out_shape=[
            jax.ShapeDtypeStruct((NP, D), jnp.float32),
            jax.ShapeDtypeStruct((NP, 1), jnp.float32),
            jax.ShapeDtypeStruct((NP, 1), jnp.float32),
            jax.ShapeDtypeStruct((NP, D), jnp.float32),
        ],
    )(aggp, aggp, u, ea, d0, d1, b, wrel, wroot, wnext)


def _att_body(r_ref, s0_ref, s1_ref, ab_ref, b_ref, g_ref, linw_ref,
              linb_ref, gout_ref, m_acc, s_acc, gxe_acc):
    p = pl.program_id(0)
    i = pl.program_id(1)
    batch = b_ref[...]
    lanes = lax.broadcasted_iota(jnp.int32, (RB, G), 1)
    mask = batch == lanes
    x = r_ref[...] + s0_ref[...] + s1_ref[...] + ab_ref[...]

    @pl.when(jnp.logical_and(p == 0, i == 0))
    def _():
        m_acc[...] = jnp.full((1, G), NEG, jnp.float32)

    @pl.when(p == 0)
    def _():
        xb = jnp.where(mask, x, NEG)
        m_acc[...] = jnp.maximum(m_acc[...], jnp.max(xb, axis=0, keepdims=True))

    @pl.when(jnp.logical_and(p == 1, i == 0))
    def _():
        s_acc[...] = jnp.zeros((1, G), jnp.float32)
        gxe_acc[...] = jnp.zeros((G, D), jnp.float32)

    @pl.when(p == 1)
    def _():
        maskf = mask.astype(jnp.float32)
        mb = jnp.sum(maskf * m_acc[...], axis=1, keepdims=True)
        e = jnp.exp(x - mb)
        s_acc[...] += jnp.sum(maskf * e, axis=0, keepdims=True)
        ge = g_ref[...] * e
        gxe_acc[...] += _dot(maskf, ge, (((0,), (0,)), ((), ())))

    @pl.when(jnp.logical_and(p == 1, i == NBLK - 1))
    def _():
        gx = gxe_acc[...] / (jnp.transpose(s_acc[...]) + 1e-16)
        gout_ref[...] = jnp.tanh(_dot(gx, linw_ref[...]) + linb_ref[...])


def _att_call(r, s0, s1, ab, batch, g, linw, linb):
    col = pl.BlockSpec((RB, 1), lambda p, i: (i, 0))
    return pl.pallas_call(
        _att_body,
        grid=(2, NBLK),
        in_specs=[
            col, col, col,
            pl.BlockSpec((1, 1), lambda p, i: (0, 0)),
            col,
            pl.BlockSpec((RB, D), lambda p, i: (i * p, 0)),
            pl.BlockSpec((D, D), lambda p, i: (0, 0)),
            pl.BlockSpec((1, D), lambda p, i: (0, 0)),
        ],
        out_specs=pl.BlockSpec((G, D), lambda p, i: (0, 0)),
        out_shape=jax.ShapeDtypeStruct((G, D), jnp.float32),
        scratch_shapes=[
            pltpu.VMEM((1, G), jnp.float32),
            pltpu.VMEM((1, G), jnp.float32),
            pltpu.VMEM((G, D), jnp.float32),
        ],
    )(r, s0, s1, ab, batch, g, linw, linb)


def _final_body(b_ref, g0_ref, g1_ref, g2_ref, go0_ref, go1_ref, go2_ref,
                at_ref, ab_ref, out_ref):
    ws = []
    for j, go in enumerate((go0_ref, go1_ref, go2_ref)):
        w = jnp.sum(go[...] * at_ref[j:j + 1, :], axis=1, keepdims=True)
        ws.append(w + ab_ref[j:j + 1, 0:1])
    wm = jnp.maximum(jnp.maximum(ws[0], ws[1]), ws[2])
    es = [jnp.exp(w - wm) for w in ws]
    tot = es[0] + es[1] + es[2]
    batch = b_ref[...]
    lanes = lax.broadcasted_iota(jnp.int32, (RB, G), 1)
    maskf = (batch == lanes).astype(jnp.float32)
    out = jnp.zeros((RB, D), jnp.float32)
    for sc_g, g_ref in zip(es, (g0_ref, g1_ref, g2_ref)):
        sb = _dot(maskf, sc_g / tot)
        out = out + g_ref[...] * sb
    out_ref[...] = out


def _final_call(batch, g0, g1, g2, go0, go1, go2, at, ab):
    blk = pl.BlockSpec((RB, D), lambda i: (i, 0))
    gob = pl.BlockSpec((G, D), lambda i: (0, 0))
    return pl.pallas_call(
        _final_body,
        grid=(NBLK,),
        in_specs=[
            pl.BlockSpec((RB, 1), lambda i: (i, 0)),
            blk, blk, blk, gob, gob, gob,
            pl.BlockSpec((NITER, D), lambda i: (0, 0)),
            pl.BlockSpec((NITER, 1), lambda i: (0, 0)),
        ],
        out_specs=blk,
        out_shape=jax.ShapeDtypeStruct((NP, D), jnp.float32),
    )(batch, g0, g1, g2, go0, go1, go2, at, ab)


def kernel(edge_attr, line_graph_edge_index, edge_index_batch, gcn_W, gcn_b,
           att_W_root, att_W_rel, att_b, lin_gout_W, lin_gout_b, a, a_bias):
    src = line_graph_edge_index[0].astype(jnp.int32).reshape(E // K, K)
    dst = line_graph_edge_index[1].astype(jnp.int32).reshape(E // K, K)
    ea = jnp.pad(edge_attr, ((0, NP - N), (0, 0)))
    batch = jnp.pad(edge_index_batch.astype(jnp.int32), (0, NP - N),
                    constant_values=G + 7).reshape(NP, 1)
    zrow128 = jnp.zeros((ZR, D), jnp.float32)
    zrow1 = jnp.zeros((ZR,), jnp.float32)
    ones1 = jnp.ones((K,), jnp.float32)

    degp = _sc_deg(ones1, dst, zrow1)
    d0 = degp[0].reshape(NP, 1)
    d1 = degp[1].reshape(NP, 1)

    at = jnp.transpose(a[0])
    ab = jnp.transpose(a_bias[0])
    attb = att_b.reshape(1, 1)

    u = _pre_call(ea, gcn_W[0], d0, d1)
    aggp = _sc_rows(u, src, dst, zrow128)
    gs = []
    gouts = []
    for i in range(NITER):
        g, y, r, u = _post_call(aggp, u, ea, d0, d1, gcn_b[i].reshape(1, D),
                                att_W_rel, att_W_root,
                                gcn_W[(i + 1) % NITER])
        if i < NITER - 1:
            aggp, sp = _make_sc_combo()(u, y.reshape(NP), src, dst,
                                        zrow128, zrow1)
        else:
            sp = _sc_scalar(y.reshape(NP), src, dst, zrow1)
        gout = _att_call(r, sp[0].reshape(NP, 1), sp[1].reshape(NP, 1),
                         attb, batch, g, lin_gout_W,
                         lin_gout_b.reshape(1, D))
        gs.append(g)
        gouts.append(gout)

    out = _final_call(batch, gs[0], gs[1], gs[2], gouts[0], gouts[1],
                      gouts[2], at, ab)
    return out[:N]


# speedup vs baseline: 1.0557x; 1.0557x over previous
import functools

import jax
import jax.numpy as jnp
from jax import lax
from jax.experimental import pallas as pl
from jax.experimental.pallas import tpu as pltpu
from jax.experimental.pallas import tpu_sc as plsc

N = 10000
E = 320000
D = 128
NITER = 3
G = 128

NP = 10240
RB = 1024
NBLK = NP // RB

NSC = 2
NTILE = 16
K = 64
EP = 327680
TPW = EP // (NSC * NTILE)
CH = TPW // K
ZR = NP // NTILE

NEG = -1e30
_P = lax.Precision.HIGHEST


def _dot(x, y, dn=None):
    if dn is None:
        return lax.dot(x, y, precision=_P, preferred_element_type=jnp.float32)
    return lax.dot_general(x, y, dn, precision=_P,
                           preferred_element_type=jnp.float32)


def _make_sc_segsum(width, gather):
    mesh = plsc.VectorSubcoreMesh(core_axis_name="c", subcore_axis_name="s",
                                  num_cores=NSC, num_subcores=NTILE)
    if width is None:
        row_shape = (K,)
        acc_words = NP
        out_shape = (NSC, NP)
        nbuf = 4
        ib = CH
    else:
        row_shape = (K, width)
        acc_words = NP * width
        out_shape = (NSC, NP, width)
        nbuf = 4
        ib = 40
    nib = CH // ib
    acc_shape = (NP,) if width is None else (NP, width)
    scratch = [
        pltpu.VMEM((ib, K), jnp.int32),
        *[pltpu.VMEM(row_shape, jnp.float32) for _ in range(nbuf)],
        pltpu.VMEM_SHARED(acc_shape, jnp.float32),
        pltpu.SemaphoreType.DMA((nbuf,)),
        pltpu.SemaphoreType.DMA((nbuf,)),
    ]
    if gather:
        scratch.insert(0, pltpu.VMEM((ib, K), jnp.int32))

    @functools.partial(
        pl.kernel,
        out_type=jax.ShapeDtypeStruct(out_shape, jnp.float32),
        mesh=mesh,
        scratch_types=scratch,
    )
    def kern(*refs):
        if gather:
            (table_hbm, src_hbm, dst_hbm, zrow_hbm, out_hbm,
             sidx_v, didx_v, *rest) = refs
        else:
            (ones_hbm, dst_hbm, zrow_hbm, out_hbm, didx_v, *rest) = refs
        bufs = rest[:nbuf]
        acc_sh, gsem, ssem = rest[nbuf:]
        cid = lax.axis_index("c")
        sid = lax.axis_index("s")
        wid = cid * NTILE + sid
        if not gather:
            for bb in range(nbuf):
                pltpu.sync_copy(ones_hbm, bufs[bb])
        pltpu.sync_copy(zrow_hbm, acc_sh.at[pl.ds(sid * ZR, ZR)])
        plsc.subcore_barrier()

        def stage_idx(nb):
            off = wid * CH + nb * ib
            pltpu.sync_copy(dst_hbm.at[pl.ds(off, ib)], didx_v)
            if gather:
                pltpu.sync_copy(src_hbm.at[pl.ds(off, ib)], sidx_v)

        def start_gather(j, b):
            return pltpu.async_copy(table_hbm.at[sidx_v.at[j]], bufs[b],
                                    gsem.at[b])

        def start_scatter(j, b):
            return pltpu.async_copy(bufs[b], acc_sh.at[didx_v.at[j]],
                                    ssem.at[b], add=True)

        def wait_gather(b):
            pltpu.make_async_copy(table_hbm.at[sidx_v.at[0]], bufs[b],
                                  gsem.at[b]).wait()

        def wait_scatter(b):
            pltpu.make_async_copy(bufs[b], acc_sh.at[didx_v.at[0]],
                                  ssem.at[b]).wait()

        if gather and width is not None:
            def blk(nb, carry):
                stage_idx(nb)
                for bb in range(nbuf):
                    start_gather(bb, bb)

                def body(g, c):
                    j0 = g * nbuf
                    for bb in range(nbuf):
                        wait_gather(bb)
                        start_scatter(j0 + bb, bb)
                    for bb in range(nbuf):
                        wait_scatter(bb)
                        start_gather(jnp.minimum(j0 + nbuf + bb, ib - 1), bb)
                    return c

                lax.fori_loop(0, ib // nbuf, body, 0)
                for bb in range(nbuf):
                    wait_gather(bb)
                return carry

            lax.fori_loop(0, nib, blk, 0)
        elif gather:
            stage_idx(0)
            for bb in range(nbuf):
                start_gather(bb, bb)

            def body(g, carry):
                j0 = g * nbuf
                for bb in range(nbuf):
                    wait_gather(bb)
                    start_scatter(j0 + bb, bb)
                for bb in range(nbuf):
                    wait_scatter(bb)
                    start_gather(jnp.minimum(j0 + nbuf + bb, CH - 1), bb)
                return carry

            lax.fori_loop(0, CH // nbuf, body, 0)
            for bb in range(nbuf):
                wait_gather(bb)
        else:
            stage_idx(0)

            def body(g, carry):
                j0 = g * nbuf
                for bb in range(nbuf):
                    start_scatter(j0 + bb, bb)
                for bb in range(nbuf):
                    wait_scatter(bb)
                return carry

            lax.fori_loop(0, CH // nbuf, body, 0)
        plsc.subcore_barrier()
        pltpu.sync_copy(acc_sh.at[pl.ds(sid * ZR, ZR)],
                        out_hbm.at[cid, pl.ds(sid * ZR, ZR)])

    return kern


def _make_sc_combo():
    mesh = plsc.VectorSubcoreMesh(core_axis_name="c", subcore_axis_name="s",
                                  num_cores=NSC, num_subcores=NTILE)
    nbuf = 4
    ib = 40
    nib = CH // ib
    scratch = [
        pltpu.VMEM((ib, K), jnp.int32),
        pltpu.VMEM((ib, K), jnp.int32),
        *[pltpu.VMEM((K, D), jnp.float32) for _ in range(nbuf)],
        *[pltpu.VMEM((K,), jnp.float32) for _ in range(nbuf)],
        pltpu.VMEM_SHARED((NP, D), jnp.float32),
        pltpu.VMEM_SHARED((NP,), jnp.float32),
        pltpu.SemaphoreType.DMA((nbuf,)),
        pltpu.SemaphoreType.DMA((nbuf,)),
        pltpu.SemaphoreType.DMA((nbuf,)),
        pltpu.SemaphoreType.DMA((nbuf,)),
    ]

    @functools.partial(
        pl.kernel,
        out_type=(jax.ShapeDtypeStruct((NSC, NP, D), jnp.float32),
                  jax.ShapeDtypeStruct((NSC, NP), jnp.float32)),
        mesh=mesh,
        scratch_types=scratch,
    )
    def kern(u_hbm, y_hbm, src_hbm, dst_hbm, zrow_hbm, zs_hbm,
             outr_hbm, outs_hbm, sidx_v, didx_v, *rest):
        rbufs = rest[:nbuf]
        sbufs = rest[nbuf:2 * nbuf]
        accr, accs, gsr, ssr, gss, sss = rest[2 * nbuf:]
        cid = lax.axis_index("c")
        sid = lax.axis_index("s")
        wid = cid * NTILE + sid
        pltpu.sync_copy(zrow_hbm, accr.at[pl.ds(sid * ZR, ZR)])
        pltpu.sync_copy(zs_hbm, accs.at[pl.ds(sid * ZR, ZR)])
        plsc.subcore_barrier()

        def gath_r(j, b):
            pltpu.async_copy(u_hbm.at[sidx_v.at[j]], rbufs[b], gsr.at[b])

        def scat_r(j, b):
            pltpu.async_copy(rbufs[b], accr.at[didx_v.at[j]], ssr.at[b],
                             add=True)

        def gath_s(j, b):
            pltpu.async_copy(y_hbm.at[sidx_v.at[j]], sbufs[b], gss.at[b])

        def scat_s(j, b):
            pltpu.async_copy(sbufs[b], accs.at[didx_v.at[j]], sss.at[b],
                             add=True)

        def wg_r(b):
            pltpu.make_async_copy(u_hbm.at[sidx_v.at[0]], rbufs[b],
                                  gsr.at[b]).wait()

        def ws_r(b):
            pltpu.make_async_copy(rbufs[b], accr.at[didx_v.at[0]],
                                  ssr.at[b]).wait()

        def wg_s(b):
            pltpu.make_async_copy(y_hbm.at[sidx_v.at[0]], sbufs[b],
                                  gss.at[b]).wait()

        def ws_s(b):
            pltpu.make_async_copy(sbufs[b], accs.at[didx_v.at[0]],
                                  sss.at[b]).wait()

        def blk(nb, carry):
            off = wid * CH + nb * ib
            pltpu.sync_copy(dst_hbm.at[pl.ds(off, ib)], didx_v)
            pltpu.sync_copy(src_hbm.at[pl.ds(off, ib)], sidx_v)
            for bb in range(nbuf):
                gath_r(bb, bb)
                gath_s(bb, bb)

            def body(g, c):
                j0 = g * nbuf
                for bb in range(nbuf):
                    wg_r(bb)
                    scat_r(j0 + bb, bb)
                    wg_s(bb)
                    scat_s(j0 + bb, bb)
                for bb in range(nbuf):
                    ws_r(bb)
                    gath_r(jnp.minimum(j0 + nbuf + bb, ib - 1), bb)
                    ws_s(bb)
                    gath_s(jnp.minimum(j0 + nbuf + bb, ib - 1), bb)
                return c

            lax.fori_loop(0, ib // nbuf, body, 0)
            for bb in range(nbuf):
                wg_r(bb)
                wg_s(bb)
            return carry

        lax.fori_loop(0, nib, blk, 0)
        plsc.subcore_barrier()
        pltpu.sync_copy(accr.at[pl.ds(sid * ZR, ZR)],
                        outr_hbm.at[cid, pl.ds(sid * ZR, ZR)])
        pltpu.sync_copy(accs.at[pl.ds(sid * ZR, ZR)],
                        outs_hbm.at[cid, pl.ds(sid * ZR, ZR)])

    return kern


_make_sc_combo = functools.lru_cache(maxsize=None)(_make_sc_combo)


_make_sc_segsum = functools.lru_cache(maxsize=None)(_make_sc_segsum)


def _sc_rows(table, src, dst, zrow):
    return _make_sc_segsum(D, True)(table, src, dst, zrow)


def _sc_scalar(table, src, dst, zrow):
    return _make_sc_segsum(None, True)(table, src, dst, zrow)


def _sc_deg(ones, dst, zrow):
    return _make_sc_segsum(None, False)(ones, dst, zrow)


def _pre_body(g_ref, w_ref, d0_ref, d1_ref, u_ref):
    deg = d0_ref[...] + d1_ref[...] + 1.0
    dis = lax.rsqrt(deg)
    u_ref[...] = _dot(g_ref[...], w_ref[...]) * dis


def _pre_call(g, w, d0, d1):
    return pl.pallas_call(
        _pre_body,
        grid=(NBLK,),
        in_specs=[
            pl.BlockSpec((RB, D), lambda i: (i, 0)),
            pl.BlockSpec((D, D), lambda i: (0, 0)),
            pl.BlockSpec((RB, 1), lambda i: (i, 0)),
            pl.BlockSpec((RB, 1), lambda i: (i, 0)),
        ],
        out_specs=pl.BlockSpec((RB, D), lambda i: (i, 0)),
        out_shape=jax.ShapeDtypeStruct((NP, D), jnp.float32),
    )(g, w, d0, d1)


def _post_body(a0_ref, a1_ref, u_ref, ea_ref, d0_ref, d1_ref, b_ref,
               wrel_ref, wroot_ref, wnext_ref, g_ref, y_ref, r_ref, un_ref):
    deg = d0_ref[...] + d1_ref[...] + 1.0
    dis = lax.rsqrt(deg)
    agg = a0_ref[0] + a1_ref[0]
    g = ea_ref[...] + dis * (agg + u_ref[...]) + b_ref[...]
    g = jnp.maximum(g, 0.0)
    rows = pl.program_id(0) * RB + lax.broadcasted_iota(jnp.int32, (RB, 1), 0)
    g = jnp.where(rows < N, g, 0.0)
    g_ref[...] = g
    y_ref[...] = _dot(g, wrel_ref[...])
    r_ref[...] = _dot(g, wroot_ref[...])
    un_ref[...] = _dot(g, wnext_ref[...]) * dis


def _post_call(aggp, u, ea, d0, d1, b, wrel, wroot, wnext):
    blk = pl.BlockSpec((RB, D), lambda i: (i, 0))
    col = pl.BlockSpec((RB, 1), lambda i: (i, 0))
    return pl.pallas_call(
        _post_body,
        grid=(NBLK,),
        in_specs=[
            pl.BlockSpec((1, RB, D), lambda i: (0, i, 0)),
            pl.BlockSpec((1, RB, D), lambda i: (1, i, 0)),
            blk, blk, col, col,
            pl.BlockSpec((1, D), lambda i: (0, 0)),
            pl.BlockSpec((D, 1), lambda i: (0, 0)),
            pl.BlockSpec((D, 1), lambda i: (0, 0)),
            pl.BlockSpec((D, D), lambda i: (0, 0)),
        ],
        out_specs=[blk, col, col, blk],
        out_shape=[
            jax.ShapeDtypeStruct((NP, D), jnp.float32),
            jax.ShapeDtypeStruct((NP, 1), jnp.float32),
            jax.ShapeDtypeStruct((NP, 1), jnp.float32),
            jax.ShapeDtypeStruct((NP, D), jnp.float32),
        ],
    )(aggp, aggp, u, ea, d0, d1, b, wrel, wroot, wnext)


def _att_body(r_ref, s0_ref, s1_ref, ab_ref, b_ref, g_ref, linw_ref,
              linb_ref, gout_ref, m_acc, s_acc, gxe_acc):
    p = pl.program_id(0)
    i = pl.program_id(1)
    batch = b_ref[...]
    lanes = lax.broadcasted_iota(jnp.int32, (RB, G), 1)
    mask = batch == lanes
    x = r_ref[...] + s0_ref[...] + s1_ref[...] + ab_ref[...]

    @pl.when(jnp.logical_and(p == 0, i == 0))
    def _():
        m_acc[...] = jnp.full((1, G), NEG, jnp.float32)

    @pl.when(p == 0)
    def _():
        xb = jnp.where(mask, x, NEG)
        m_acc[...] = jnp.maximum(m_acc[...], jnp.max(xb, axis=0, keepdims=True))

    @pl.when(jnp.logical_and(p == 1, i == 0))
    def _():
        s_acc[...] = jnp.zeros((1, G), jnp.float32)
        gxe_acc[...] = jnp.zeros((G, D), jnp.float32)

    @pl.when(p == 1)
    def _():
        maskf = mask.astype(jnp.float32)
        mb = jnp.sum(maskf * m_acc[...], axis=1, keepdims=True)
        e = jnp.exp(x - mb)
        s_acc[...] += jnp.sum(maskf * e, axis=0, keepdims=True)
        ge = g_ref[...] * e
        gxe_acc[...] += _dot(maskf, ge, (((0,), (0,)), ((), ())))

    @pl.when(jnp.logical_and(p == 1, i == NBLK - 1))
    def _():
        gx = gxe_acc[...] / (jnp.transpose(s_acc[...]) + 1e-16)
        gout_ref[...] = jnp.tanh(_dot(gx, linw_ref[...]) + linb_ref[...])


def _att_call(r, s0, s1, ab, batch, g, linw, linb):
    col = pl.BlockSpec((RB, 1), lambda p, i: (i, 0))
    return pl.pallas_call(
        _att_body,
        grid=(2, NBLK),
        in_specs=[
            col, col, col,
            pl.BlockSpec((1, 1), lambda p, i: (0, 0)),
            col,
            pl.BlockSpec((RB, D), lambda p, i: (i * p, 0)),
            pl.BlockSpec((D, D), lambda p, i: (0, 0)),
            pl.BlockSpec((1, D), lambda p, i: (0, 0)),
        ],
        out_specs=pl.BlockSpec((G, D), lambda p, i: (0, 0)),
        out_shape=jax.ShapeDtypeStruct((G, D), jnp.float32),
        scratch_shapes=[
            pltpu.VMEM((1, G), jnp.float32),
            pltpu.VMEM((1, G), jnp.float32),
            pltpu.VMEM((G, D), jnp.float32),
        ],
    )(r, s0, s1, ab, batch, g, linw, linb)


def _final_body(b_ref, g0_ref, g1_ref, g2_ref, go0_ref, go1_ref, go2_ref,
                at_ref, ab_ref, out_ref):
    ws = []
    for j, go in enumerate((go0_ref, go1_ref, go2_ref)):
        w = jnp.sum(go[...] * at_ref[j:j + 1, :], axis=1, keepdims=True)
        ws.append(w + ab_ref[j:j + 1, 0:1])
    wm = jnp.maximum(jnp.maximum(ws[0], ws[1]), ws[2])
    es = [jnp.exp(w - wm) for w in ws]
    tot = es[0] + es[1] + es[2]
    batch = b_ref[...]
    lanes = lax.broadcasted_iota(jnp.int32, (RB, G), 1)
    maskf = (batch == lanes).astype(jnp.float32)
    out = jnp.zeros((RB, D), jnp.float32)
    for sc_g, g_ref in zip(es, (g0_ref, g1_ref, g2_ref)):
        sb = _dot(maskf, sc_g / tot)
        out = out + g_ref[...] * sb
    out_ref[...] = out


def _final_call(batch, g0, g1, g2, go0, go1, go2, at, ab):
    blk = pl.BlockSpec((RB, D), lambda i: (i, 0))
    gob = pl.BlockSpec((G, D), lambda i: (0, 0))
    return pl.pallas_call(
        _final_body,
        grid=(NBLK,),
        in_specs=[
            pl.BlockSpec((RB, 1), lambda i: (i, 0)),
            blk, blk, blk, gob, gob, gob,
            pl.BlockSpec((NITER, D), lambda i: (0, 0)),
            pl.BlockSpec((NITER, 1), lambda i: (0, 0)),
        ],
        out_specs=blk,
        out_shape=jax.ShapeDtypeStruct((NP, D), jnp.float32),
    )(batch, g0, g1, g2, go0, go1, go2, at, ab)


def kernel(edge_attr, line_graph_edge_index, edge_index_batch, gcn_W, gcn_b,
           att_W_root, att_W_rel, att_b, lin_gout_W, lin_gout_b, a, a_bias):
    pad = N + (jnp.arange(EP - E, dtype=jnp.int32) % (NP - N))
    src = jnp.concatenate(
        [line_graph_edge_index[0].astype(jnp.int32), pad]).reshape(EP // K, K)
    dst = jnp.concatenate(
        [line_graph_edge_index[1].astype(jnp.int32), pad]).reshape(EP // K, K)
    ea = jnp.pad(edge_attr, ((0, NP - N), (0, 0)))
    batch = jnp.pad(edge_index_batch.astype(jnp.int32), (0, NP - N),
                    constant_values=G + 7).reshape(NP, 1)
    zrow128 = jnp.zeros((ZR, D), jnp.float32)
    zrow1 = jnp.zeros((ZR,), jnp.float32)
    ones1 = jnp.ones((K,), jnp.float32)

    degp = _sc_deg(ones1, dst, zrow1)
    d0 = degp[0].reshape(NP, 1)
    d1 = degp[1].reshape(NP, 1)

    at = jnp.transpose(a[0])
    ab = jnp.transpose(a_bias[0])
    attb = att_b.reshape(1, 1)

    u = _pre_call(ea, gcn_W[0], d0, d1)
    aggp = _sc_rows(u, src, dst, zrow128)
    gs = []
    gouts = []
    for i in range(NITER):
        g, y, r, u = _post_call(aggp, u, ea, d0, d1, gcn_b[i].reshape(1, D),
                                att_W_rel, att_W_root,
                                gcn_W[(i + 1) % NITER])
        if i < NITER - 1:
            aggp, sp = _make_sc_combo()(u, y.reshape(NP), src, dst,
                                        zrow128, zrow1)
        else:
            sp = _sc_scalar(y.reshape(NP), src, dst, zrow1)
        gout = _att_call(r, sp[0].reshape(NP, 1), sp[1].reshape(NP, 1),
                         attb, batch, g, lin_gout_W,
                         lin_gout_b.reshape(1, D))
        gs.append(g)
        gouts.append(gout)

    out = _final_call(batch, gs[0], gs[1], gs[2], gouts[0], gouts[1],
                      gouts[2], at, ab)
    return out[:N]


# speedup vs baseline: 1.0804x; 1.0234x over previous
import functools

import jax
import jax.numpy as jnp
from jax import lax
from jax.experimental import pallas as pl
from jax.experimental.pallas import tpu as pltpu
from jax.experimental.pallas import tpu_sc as plsc

N = 10000
E = 320000
D = 128
NITER = 3
G = 128

NP = 10240
RB = 1024
NBLK = NP // RB

NSC = 2
NTILE = 16
K = 64
EP = 327680
TPW = EP // (NSC * NTILE)
CH = TPW // K
ZR = NP // NTILE

NEG = -1e30
_P = lax.Precision.HIGHEST


def _dot(x, y, dn=None):
    if dn is None:
        return lax.dot(x, y, precision=_P, preferred_element_type=jnp.float32)
    return lax.dot_general(x, y, dn, precision=_P,
                           preferred_element_type=jnp.float32)


def _make_sc_segsum(width, gather):
    mesh = plsc.VectorSubcoreMesh(core_axis_name="c", subcore_axis_name="s",
                                  num_cores=NSC, num_subcores=NTILE)
    if width is None:
        row_shape = (K,)
        acc_words = NP
        out_shape = (NSC, NP)
        nbuf = 4
        ib = CH
    else:
        row_shape = (K, width)
        acc_words = NP * width
        out_shape = (NSC, NP, width)
        nbuf = 4
        ib = 40
    nib = CH // ib
    acc_shape = (NP,) if width is None else (NP, width)
    scratch = [
        pltpu.VMEM((ib, K), jnp.int32),
        *[pltpu.VMEM(row_shape, jnp.float32) for _ in range(nbuf)],
        pltpu.VMEM_SHARED(acc_shape, jnp.float32),
        pltpu.SemaphoreType.DMA((nbuf,)),
        pltpu.SemaphoreType.DMA((nbuf,)),
    ]
    if gather:
        scratch.insert(0, pltpu.VMEM((ib, K), jnp.int32))

    @functools.partial(
        pl.kernel,
        out_type=jax.ShapeDtypeStruct(out_shape, jnp.float32),
        mesh=mesh,
        scratch_types=scratch,
    )
    def kern(*refs):
        if gather:
            (table_hbm, src_hbm, dst_hbm, zrow_hbm, out_hbm,
             sidx_v, didx_v, *rest) = refs
        else:
            (ones_hbm, dst_hbm, zrow_hbm, out_hbm, didx_v, *rest) = refs
        bufs = rest[:nbuf]
        acc_sh, gsem, ssem = rest[nbuf:]
        cid = lax.axis_index("c")
        sid = lax.axis_index("s")
        wid = cid * NTILE + sid
        if not gather:
            for bb in range(nbuf):
                pltpu.sync_copy(ones_hbm, bufs[bb])
        pltpu.sync_copy(zrow_hbm, acc_sh.at[pl.ds(sid * ZR, ZR)])
        plsc.subcore_barrier()

        def stage_idx(nb):
            off = wid * CH + nb * ib
            pltpu.sync_copy(dst_hbm.at[pl.ds(off, ib)], didx_v)
            if gather:
                pltpu.sync_copy(src_hbm.at[pl.ds(off, ib)], sidx_v)

        def start_gather(j, b):
            return pltpu.async_copy(table_hbm.at[sidx_v.at[j]], bufs[b],
                                    gsem.at[b])

        def start_scatter(j, b):
            return pltpu.async_copy(bufs[b], acc_sh.at[didx_v.at[j]],
                                    ssem.at[b], add=True)

        def wait_gather(b):
            pltpu.make_async_copy(table_hbm.at[sidx_v.at[0]], bufs[b],
                                  gsem.at[b]).wait()

        def wait_scatter(b):
            pltpu.make_async_copy(bufs[b], acc_sh.at[didx_v.at[0]],
                                  ssem.at[b]).wait()

        if gather and width is not None:
            def blk(nb, carry):
                stage_idx(nb)
                for bb in range(nbuf):
                    start_gather(bb, bb)

                def body(g, c):
                    j0 = g * nbuf
                    for bb in range(nbuf):
                        wait_gather(bb)
                        start_scatter(j0 + bb, bb)
                    for bb in range(nbuf):
                        wait_scatter(bb)
                        start_gather(jnp.minimum(j0 + nbuf + bb, ib - 1), bb)
                    return c

                lax.fori_loop(0, ib // nbuf, body, 0)
                for bb in range(nbuf):
                    wait_gather(bb)
                return carry

            lax.fori_loop(0, nib, blk, 0)
        elif gather:
            stage_idx(0)
            for bb in range(nbuf):
                start_gather(bb, bb)

            def body(g, carry):
                j0 = g * nbuf
                for bb in range(nbuf):
                    wait_gather(bb)
                    start_scatter(j0 + bb, bb)
                for bb in range(nbuf):
                    wait_scatter(bb)
                    start_gather(jnp.minimum(j0 + nbuf + bb, CH - 1), bb)
                return carry

            lax.fori_loop(0, CH // nbuf, body, 0)
            for bb in range(nbuf):
                wait_gather(bb)
        else:
            stage_idx(0)

            def body(g, carry):
                j0 = g * nbuf
                for bb in range(nbuf):
                    start_scatter(j0 + bb, bb)
                for bb in range(nbuf):
                    wait_scatter(bb)
                return carry

            lax.fori_loop(0, CH // nbuf, body, 0)
        plsc.subcore_barrier()
        pltpu.sync_copy(acc_sh.at[pl.ds(sid * ZR, ZR)],
                        out_hbm.at[cid, pl.ds(sid * ZR, ZR)])

    return kern


def _make_sc_combo():
    mesh = plsc.VectorSubcoreMesh(core_axis_name="c", subcore_axis_name="s",
                                  num_cores=NSC, num_subcores=NTILE)
    nbuf = 4
    ib = 40
    nib = CH // ib
    scratch = [
        pltpu.VMEM((ib, K), jnp.int32),
        pltpu.VMEM((ib, K), jnp.int32),
        *[pltpu.VMEM((K, D), jnp.float32) for _ in range(nbuf)],
        *[pltpu.VMEM((K,), jnp.float32) for _ in range(nbuf)],
        pltpu.VMEM_SHARED((NP, D), jnp.float32),
        pltpu.VMEM_SHARED((NP,), jnp.float32),
        pltpu.SemaphoreType.DMA((nbuf,)),
        pltpu.SemaphoreType.DMA((nbuf,)),
        pltpu.SemaphoreType.DMA((nbuf,)),
        pltpu.SemaphoreType.DMA((nbuf,)),
    ]

    @functools.partial(
        pl.kernel,
        out_type=(jax.ShapeDtypeStruct((NSC, NP, D), jnp.float32),
                  jax.ShapeDtypeStruct((NSC, NP), jnp.float32)),
        mesh=mesh,
        scratch_types=scratch,
    )
    def kern(u_hbm, y_hbm, src_hbm, dst_hbm, zrow_hbm, zs_hbm,
             outr_hbm, outs_hbm, sidx_v, didx_v, *rest):
        rbufs = rest[:nbuf]
        sbufs = rest[nbuf:2 * nbuf]
        accr, accs, gsr, ssr, gss, sss = rest[2 * nbuf:]
        cid = lax.axis_index("c")
        sid = lax.axis_index("s")
        wid = cid * NTILE + sid
        pltpu.sync_copy(zrow_hbm, accr.at[pl.ds(sid * ZR, ZR)])
        pltpu.sync_copy(zs_hbm, accs.at[pl.ds(sid * ZR, ZR)])
        plsc.subcore_barrier()

        def gath_r(j, b):
            pltpu.async_copy(u_hbm.at[sidx_v.at[j]], rbufs[b], gsr.at[b])

        def scat_r(j, b):
            pltpu.async_copy(rbufs[b], accr.at[didx_v.at[j]], ssr.at[b],
                             add=True)

        def gath_s(j, b):
            pltpu.async_copy(y_hbm.at[sidx_v.at[j]], sbufs[b], gss.at[b])

        def scat_s(j, b):
            pltpu.async_copy(sbufs[b], accs.at[didx_v.at[j]], sss.at[b],
                             add=True)

        def wg_r(b):
            pltpu.make_async_copy(u_hbm.at[sidx_v.at[0]], rbufs[b],
                                  gsr.at[b]).wait()

        def ws_r(b):
            pltpu.make_async_copy(rbufs[b], accr.at[didx_v.at[0]],
                                  ssr.at[b]).wait()

        def wg_s(b):
            pltpu.make_async_copy(y_hbm.at[sidx_v.at[0]], sbufs[b],
                                  gss.at[b]).wait()

        def ws_s(b):
            pltpu.make_async_copy(sbufs[b], accs.at[didx_v.at[0]],
                                  sss.at[b]).wait()

        def blk(nb, carry):
            off = wid * CH + nb * ib
            pltpu.sync_copy(dst_hbm.at[pl.ds(off, ib)], didx_v)
            pltpu.sync_copy(src_hbm.at[pl.ds(off, ib)], sidx_v)
            for bb in range(nbuf):
                gath_r(bb, bb)
                gath_s(bb, bb)

            def body(g, c):
                j0 = g * nbuf
                for bb in range(nbuf):
                    wg_r(bb)
                    scat_r(j0 + bb, bb)
                    wg_s(bb)
                    scat_s(j0 + bb, bb)
                for bb in range(nbuf):
                    ws_r(bb)
                    gath_r(jnp.minimum(j0 + nbuf + bb, ib - 1), bb)
                    ws_s(bb)
                    gath_s(jnp.minimum(j0 + nbuf + bb, ib - 1), bb)
                return c

            lax.fori_loop(0, ib // nbuf, body, 0)
            for bb in range(nbuf):
                wg_r(bb)
                wg_s(bb)
            return carry

        lax.fori_loop(0, nib, blk, 0)
        plsc.subcore_barrier()
        pltpu.sync_copy(accr.at[pl.ds(sid * ZR, ZR)],
                        outr_hbm.at[cid, pl.ds(sid * ZR, ZR)])
        pltpu.sync_copy(accs.at[pl.ds(sid * ZR, ZR)],
                        outs_hbm.at[cid, pl.ds(sid * ZR, ZR)])

    return kern


_make_sc_combo = functools.lru_cache(maxsize=None)(_make_sc_combo)


_make_sc_segsum = functools.lru_cache(maxsize=None)(_make_sc_segsum)


def _sc_rows(table, src, dst, zrow):
    return _make_sc_segsum(D, True)(table, src, dst, zrow)


def _sc_scalar(table, src, dst, zrow):
    return _make_sc_segsum(None, True)(table, src, dst, zrow)


def _sc_deg(ones, dst, zrow):
    return _make_sc_segsum(None, False)(ones, dst, zrow)


def _pre_body(g_ref, w_ref, d0_ref, d1_ref, u_ref):
    deg = d0_ref[...] + d1_ref[...] + 1.0
    dis = lax.rsqrt(deg)
    u_ref[...] = _dot(g_ref[...], w_ref[...]) * dis


def _pre_call(g, w, d0, d1):
    return pl.pallas_call(
        _pre_body,
        grid=(NBLK,),
        in_specs=[
            pl.BlockSpec((RB, D), lambda i: (i, 0)),
            pl.BlockSpec((D, D), lambda i: (0, 0)),
            pl.BlockSpec((RB, 1), lambda i: (i, 0)),
            pl.BlockSpec((RB, 1), lambda i: (i, 0)),
        ],
        out_specs=pl.BlockSpec((RB, D), lambda i: (i, 0)),
        out_shape=jax.ShapeDtypeStruct((NP, D), jnp.float32),
    )(g, w, d0, d1)


def _post_body(a0_ref, a1_ref, u_ref, ea_ref, d0_ref, d1_ref, b_ref,
               wrel_ref, wroot_ref, wnext_ref, g_ref, y_ref, r_ref, un_ref):
    deg = d0_ref[...] + d1_ref[...] + 1.0
    dis = lax.rsqrt(deg)
    agg = a0_ref[0] + a1_ref[0]
    g = ea_ref[...] + dis * (agg + u_ref[...]) + b_ref[...]
    g = jnp.maximum(g, 0.0)
    rows = pl.program_id(0) * RB + lax.broadcasted_iota(jnp.int32, (RB, 1), 0)
    g = jnp.where(rows < N, g, 0.0)
    g_ref[...] = g
    y_ref[...] = _dot(g, wrel_ref[...])
    r_ref[...] = _dot(g, wroot_ref[...])
    un_ref[...] = _dot(g, wnext_ref[...]) * dis


def _post_call(aggp, u, ea, d0, d1, b, wrel, wroot, wnext):
    blk = pl.BlockSpec((RB, D), lambda i: (i, 0))
    col = pl.BlockSpec((RB, 1), lambda i: (i, 0))
    return pl.pallas_call(
        _post_body,
        grid=(NBLK,),
        in_specs=[
            pl.BlockSpec((1, RB, D), lambda i: (0, i, 0)),
            pl.BlockSpec((1, RB, D), lambda i: (1, i, 0)),
            blk, blk, col, col,
            pl.BlockSpec((1, D), lambda i: (0, 0)),
            pl.BlockSpec((D, 1), lambda i: (0, 0)),
            pl.BlockSpec((D, 1), lambda i: (0, 0)),
            pl.BlockSpec((D, D), lambda i: (0, 0)),
        ],
        out_specs=[blk, col, col, blk],
        out_shape=[
            jax.ShapeDtypeStruct((NP, D), jnp.float32),
            jax.ShapeDtypeStruct((NP, 1), jnp.float32),
            jax.ShapeDtypeStruct((NP, 1), jnp.float32),
            jax.ShapeDtypeStruct((NP, D), jnp.float32),
        ],
    )(aggp, aggp, u, ea, d0, d1, b, wrel, wroot, wnext)


def _att_final_body(b_ref, r0_ref, r1_ref, r2_ref,
                    s00_ref, s01_ref, s10_ref, s11_ref, s20_ref, s21_ref,
                    ab_ref, g0_ref, g1_ref, g2_ref, linw_ref, linb_ref,
                    at_ref, abias_ref, out_ref,
                    m_acc, s_acc, gxe0, gxe1, gxe2, sc0, sc1, sc2):
    p = pl.program_id(0)
    i = pl.program_id(1)
    batch = b_ref[...]
    lanes = lax.broadcasted_iota(jnp.int32, (RB, G), 1)
    mask = batch == lanes
    rs = (r0_ref, r1_ref, r2_ref)
    ss = ((s00_ref, s01_ref), (s10_ref, s11_ref), (s20_ref, s21_ref))
    gs = (g0_ref, g1_ref, g2_ref)
    gxes = (gxe0, gxe1, gxe2)
    scs = (sc0, sc1, sc2)

    def x_it(it):
        return rs[it][...] + ss[it][0][...] + ss[it][1][...] + ab_ref[...]

    @pl.when(jnp.logical_and(p == 0, i == 0))
    def _():
        m_acc[...] = jnp.full((NITER, G), NEG, jnp.float32)

    @pl.when(p == 0)
    def _():
        for it in range(NITER):
            xb = jnp.where(mask, x_it(it), NEG)
            m_acc[it:it + 1, :] = jnp.maximum(
                m_acc[it:it + 1, :], jnp.max(xb, axis=0, keepdims=True))

    @pl.when(jnp.logical_and(p == 1, i == 0))
    def _():
        s_acc[...] = jnp.zeros((NITER, G), jnp.float32)
        for it in range(NITER):
            gxes[it][...] = jnp.zeros((G, D), jnp.float32)

    @pl.when(p == 1)
    def _():
        maskf = mask.astype(jnp.float32)
        for it in range(NITER):
            mb = jnp.sum(maskf * m_acc[it:it + 1, :], axis=1, keepdims=True)
            e = jnp.exp(x_it(it) - mb)
            s_acc[it:it + 1, :] += jnp.sum(maskf * e, axis=0, keepdims=True)
            ge = gs[it][...] * e
            gxes[it][...] += _dot(maskf, ge, (((0,), (0,)), ((), ())))

    @pl.when(jnp.logical_and(p == 1, i == NBLK - 1))
    def _():
        ws = []
        for it in range(NITER):
            gx = gxes[it][...] / (
                jnp.transpose(s_acc[it:it + 1, :]) + 1e-16)
            gout = jnp.tanh(_dot(gx, linw_ref[...]) + linb_ref[...])
            w = jnp.sum(gout * at_ref[it:it + 1, :], axis=1, keepdims=True)
            ws.append(w + abias_ref[it:it + 1, 0:1])
        wm = jnp.maximum(jnp.maximum(ws[0], ws[1]), ws[2])
        es = [jnp.exp(w - wm) for w in ws]
        tot = es[0] + es[1] + es[2]
        for it in range(NITER):
            scs[it][...] = es[it] / tot

    @pl.when(p == 2)
    def _():
        maskf = mask.astype(jnp.float32)
        out = jnp.zeros((RB, D), jnp.float32)
        for it in range(NITER):
            sb = _dot(maskf, scs[it][...])
            out = out + gs[it][...] * sb
        out_ref[...] = out


def _att_final_call(batch, rs, sps, ab, gs, linw, linb, at, abias):
    col = pl.BlockSpec((RB, 1), lambda p, i: (i, 0))
    gblk = pl.BlockSpec((RB, D), lambda p, i: (jnp.where(p == 0, 0, i), 0))
    one = lambda shape: pl.BlockSpec(shape, lambda p, i: tuple(
        0 for _ in shape))
    sp_cols = []
    for sp in sps:
        sp_cols.append(sp[0].reshape(NP, 1))
        sp_cols.append(sp[1].reshape(NP, 1))
    return pl.pallas_call(
        _att_final_body,
        grid=(3, NBLK),
        in_specs=[col, col, col, col,
                  col, col, col, col, col, col,
                  one((1, 1)),
                  gblk, gblk, gblk,
                  one((D, D)), one((1, D)), one((NITER, D)),
                  one((NITER, 1))],
        out_specs=pl.BlockSpec((RB, D), lambda p, i: (i, 0)),
        out_shape=jax.ShapeDtypeStruct((NP, D), jnp.float32),
        scratch_shapes=[
            pltpu.VMEM((NITER, G), jnp.float32),
            pltpu.VMEM((NITER, G), jnp.float32),
            pltpu.VMEM((G, D), jnp.float32),
            pltpu.VMEM((G, D), jnp.float32),
            pltpu.VMEM((G, D), jnp.float32),
            pltpu.VMEM((G, 1), jnp.float32),
            pltpu.VMEM((G, 1), jnp.float32),
            pltpu.VMEM((G, 1), jnp.float32),
        ],
    )(batch, rs[0], rs[1], rs[2], *sp_cols, ab, gs[0], gs[1], gs[2],
      linw, linb, at, abias)


def kernel(edge_attr, line_graph_edge_index, edge_index_batch, gcn_W, gcn_b,
           att_W_root, att_W_rel, att_b, lin_gout_W, lin_gout_b, a, a_bias):
    pad = N + (jnp.arange(EP - E, dtype=jnp.int32) % (NP - N))
    src = jnp.concatenate(
        [line_graph_edge_index[0].astype(jnp.int32), pad]).reshape(EP // K, K)
    dst = jnp.concatenate(
        [line_graph_edge_index[1].astype(jnp.int32), pad]).reshape(EP // K, K)
    ea = jnp.pad(edge_attr, ((0, NP - N), (0, 0)))
    batch = jnp.pad(edge_index_batch.astype(jnp.int32), (0, NP - N),
                    constant_values=G + 7).reshape(NP, 1)
    zrow128 = jnp.zeros((ZR, D), jnp.float32)
    zrow1 = jnp.zeros((ZR,), jnp.float32)
    ones1 = jnp.ones((K,), jnp.float32)

    degp = _sc_deg(ones1, dst, zrow1)
    d0 = degp[0].reshape(NP, 1)
    d1 = degp[1].reshape(NP, 1)

    at = jnp.transpose(a[0])
    ab = jnp.transpose(a_bias[0])
    attb = att_b.reshape(1, 1)

    u = _pre_call(ea, gcn_W[0], d0, d1)
    aggp = _sc_rows(u, src, dst, zrow128)
    gs = []
    rs = []
    sps = []
    for i in range(NITER):
        g, y, r, u = _post_call(aggp, u, ea, d0, d1, gcn_b[i].reshape(1, D),
                                att_W_rel, att_W_root,
                                gcn_W[(i + 1) % NITER])
        if i < NITER - 1:
            aggp, sp = _make_sc_combo()(u, y.reshape(NP), src, dst,
                                        zrow128, zrow1)
        else:
            sp = _sc_scalar(y.reshape(NP), src, dst, zrow1)
        gs.append(g)
        rs.append(r)
        sps.append(sp)

    out = _att_final_call(batch, rs, sps, attb, gs, lin_gout_W,
                          lin_gout_b.reshape(1, D), at, ab)
    return out[:N]
